# baseline (speedup 1.0000x reference)
import jax
import jax.numpy as jnp
from jax import lax
from jax.experimental import pallas as pl
from jax.experimental.pallas import tpu as pltpu

_BLK = 10000
_G = 256


def _gap_kernel(h_ref, seg_ref, W1_ref, b1_ref, W2T_ref, out_ref, s_ref):
    i = pl.program_id(0)
    nblk = pl.num_programs(0)

    @pl.when(i == 0)
    def _init():
        s_ref[...] = jnp.zeros_like(s_ref)
        out_ref[...] = jnp.zeros_like(out_ref)

    hb = h_ref[...].astype(jnp.bfloat16)
    seg = seg_ref[0]

    u = jnp.tanh(
        lax.dot_general(hb, W1_ref[...], (((1,), (0,)), ((), ())),
                        preferred_element_type=jnp.float32) + b1_ref[...])
    logits = lax.dot_general(W2T_ref[...], u.astype(jnp.bfloat16),
                             (((1,), (1,)), ((), ())),
                             preferred_element_type=jnp.float32)
    bound = jnp.sum(jnp.abs(W2T_ref[...].astype(jnp.float32)),
                    axis=1, keepdims=True)
    ex = jnp.exp(logits - bound)

    gid = lax.broadcasted_iota(jnp.int32, (_G, 1), 0).astype(jnp.bfloat16)
    w = jnp.where(seg == gid, ex.astype(jnp.bfloat16),
                  jnp.bfloat16(0.0))

    out_ref[...] += lax.dot_general(w, hb, (((1,), (0,)), ((), ())),
                                    preferred_element_type=jnp.float32)
    ones = jnp.ones((_BLK, 8), jnp.bfloat16)
    s_ref[...] += lax.dot_general(w, ones, (((1,), (0,)), ((), ())),
                                  preferred_element_type=jnp.float32)[:, :1]

    @pl.when(i == nblk - 1)
    def _fin():
        s = s_ref[...]
        out_ref[...] = jnp.where(s > 0.0, out_ref[...] / s, 0.0)


def _pallas_gap(h, seg, W1, b1r, W2T, *, interpret=False):
    n, d = h.shape
    hdim = W1.shape[1]
    nblk = n // _BLK
    return pl.pallas_call(
        _gap_kernel,
        grid=(nblk,),
        in_specs=[
            pl.BlockSpec((_BLK, d), lambda i: (i, 0)),
            pl.BlockSpec((1, 1, _BLK), lambda i: (i, 0, 0)),
            pl.BlockSpec((d, hdim), lambda i: (0, 0)),
            pl.BlockSpec((1, hdim), lambda i: (0, 0)),
            pl.BlockSpec((1, hdim), lambda i: (0, 0)),
        ],
        out_specs=pl.BlockSpec((_G, d), lambda i: (0, 0)),
        out_shape=jax.ShapeDtypeStruct((_G, d), jnp.float32),
        scratch_shapes=[
            pltpu.VMEM((_G, 1), jnp.float32),
        ],
        interpret=interpret,
    )(h, seg, W1, b1r, W2T)


@jax.jit
def kernel(h, batch, W1, b1, W2, b2):
    n = h.shape[0]
    nblk = n // _BLK
    seg = batch.astype(jnp.int32).astype(jnp.bfloat16).reshape(nblk, 1, _BLK)
    del b2
    return _pallas_gap(h, seg, W1.astype(jnp.bfloat16), b1.reshape(1, -1),
                       W2.reshape(1, -1).astype(jnp.bfloat16))


# speedup vs baseline: 1.1830x; 1.1830x over previous
import jax
import jax.numpy as jnp
from jax import lax
from jax.experimental import pallas as pl
from jax.experimental.pallas import tpu as pltpu

_BLK = 10000
_G = 256


def _gap_kernel(h_ref, seg_ref, W1_ref, b1_ref, W2T_ref, out_ref, s_ref):
    i = pl.program_id(0)
    nblk = pl.num_programs(0)

    @pl.when(i == 0)
    def _init():
        s_ref[...] = jnp.zeros_like(s_ref)
        out_ref[...] = jnp.zeros_like(out_ref)

    hb = h_ref[...].astype(jnp.bfloat16)
    seg = seg_ref[0]

    u = jnp.tanh(
        lax.dot_general(hb, W1_ref[...], (((1,), (0,)), ((), ())),
                        preferred_element_type=jnp.float32) + b1_ref[...])
    logits = lax.dot_general(W2T_ref[...], u.astype(jnp.bfloat16),
                             (((1,), (1,)), ((), ())),
                             preferred_element_type=jnp.float32)
    bound = jnp.sum(jnp.abs(W2T_ref[...].astype(jnp.float32)),
                    axis=1, keepdims=True)
    ex = jnp.exp(logits - bound)

    gid = lax.broadcasted_iota(jnp.int32, (_G, 1), 0).astype(jnp.bfloat16)
    w = jnp.where(seg == gid, ex.astype(jnp.bfloat16),
                  jnp.bfloat16(0.0))

    out_ref[...] += lax.dot_general(w, hb, (((1,), (0,)), ((), ())),
                                    preferred_element_type=jnp.float32)
    s_ref[...] += jnp.sum(w.astype(jnp.float32), axis=1, keepdims=True)

    @pl.when(i == nblk - 1)
    def _fin():
        s = s_ref[...]
        out_ref[...] = jnp.where(s > 0.0, out_ref[...] / s, 0.0)


def _pallas_gap(h, seg, W1, b1r, W2T, *, interpret=False):
    n, d = h.shape
    hdim = W1.shape[1]
    nblk = n // _BLK
    return pl.pallas_call(
        _gap_kernel,
        grid=(nblk,),
        in_specs=[
            pl.BlockSpec((_BLK, d), lambda i: (i, 0)),
            pl.BlockSpec((1, 1, _BLK), lambda i: (i, 0, 0)),
            pl.BlockSpec((d, hdim), lambda i: (0, 0)),
            pl.BlockSpec((1, hdim), lambda i: (0, 0)),
            pl.BlockSpec((1, hdim), lambda i: (0, 0)),
        ],
        out_specs=pl.BlockSpec((_G, d), lambda i: (0, 0)),
        out_shape=jax.ShapeDtypeStruct((_G, d), jnp.float32),
        scratch_shapes=[
            pltpu.VMEM((_G, 1), jnp.float32),
        ],
        interpret=interpret,
    )(h, seg, W1, b1r, W2T)


@jax.jit
def kernel(h, batch, W1, b1, W2, b2):
    n = h.shape[0]
    nblk = n // _BLK
    seg = batch.astype(jnp.int32).astype(jnp.bfloat16).reshape(nblk, 1, _BLK)
    del b2
    return _pallas_gap(h, seg, W1.astype(jnp.bfloat16), b1.reshape(1, -1),
                       W2.reshape(1, -1).astype(jnp.bfloat16))
